# baseline (speedup 1.0000x reference)
import functools

import jax
import jax.numpy as jnp
from jax import lax
from jax.experimental import pallas as pl
from jax.experimental.pallas import tpu as pltpu
from jax.experimental.pallas import tpu_sc as plsc

_D = 128
_NC = 2
_NS = 16
_NW = _NC * _NS
_C = 400
_W = 50


@jax.jit
def _flat_gather(idx, table):
    n_rows = idx.shape[0]
    n_samp = n_rows // _W
    s_per_w = n_samp // _NW
    b_per_w = n_rows // _NW
    cs = _C // _W
    nch = b_per_w // _C
    mesh = plsc.VectorSubcoreMesh(core_axis_name="c", subcore_axis_name="s")

    @functools.partial(
        pl.kernel,
        out_type=jax.ShapeDtypeStruct((n_samp, _W, _D), jnp.float32),
        mesh=mesh,
        compiler_params=pltpu.CompilerParams(use_tc_tiling_on_sc=True),
        scratch_types=[
            pltpu.VMEM((b_per_w,), jnp.int32),
            pltpu.VMEM((2, _C, _D), jnp.float32),
            pltpu.SemaphoreType.DMA,
            pltpu.SemaphoreType.DMA,
        ],
    )
    def run(idx_hbm, table_hbm, out_hbm, idx_v, rows_v, gsem, osem):
        wid = lax.axis_index("s") * _NC + lax.axis_index("c")
        base = wid * b_per_w
        samp0 = wid * s_per_w
        pltpu.sync_copy(idx_hbm.at[pl.ds(base, b_per_w)], idx_v)

        def g_copy(ch, b):
            return pltpu.make_async_copy(
                table_hbm.at[idx_v.at[pl.ds(ch * _C, _C)]], rows_v.at[b], gsem
            )

        def o_copy(ch, b):
            return pltpu.make_async_copy(
                rows_v.at[b].reshape(cs, _W, _D),
                out_hbm.at[pl.ds(samp0 + ch * cs, cs)],
                osem,
            )

        g_copy(0, 0).start()

        @pl.loop(0, nch, step=2)
        def chunk_loop(i):
            g_copy(i, 0).wait()

            @pl.when(i > 0)
            def _():
                o_copy(i - 1, 1).wait()

            g_copy(i + 1, 1).start()
            o_copy(i, 0).start()
            g_copy(i + 1, 1).wait()
            o_copy(i, 0).wait()

            @pl.when(i + 2 < nch)
            def _():
                g_copy(i + 2, 0).start()

            o_copy(i + 1, 1).start()

        o_copy(nch - 1, 1).wait()

    return run(idx, table)


def kernel(x, embedding):
    idx = x.reshape(x.shape[0] * x.shape[1]).astype(jnp.int32)
    return _flat_gather(idx, embedding)


# speedup vs baseline: 1.8258x; 1.8258x over previous
import functools

import jax
import jax.numpy as jnp
from jax import lax
from jax.experimental import pallas as pl
from jax.experimental.pallas import tpu as pltpu
from jax.experimental.pallas import tpu_sc as plsc

_D = 128
_NC = 2
_NS = 16
_NW = _NC * _NS
_C = 200
_NB = 4


@jax.jit
def _flat_gather(idx, table):
    n_rows = idx.shape[0]
    b_per_w = n_rows // _NW
    nch = b_per_w // _C
    mesh = plsc.VectorSubcoreMesh(core_axis_name="c", subcore_axis_name="s")

    @functools.partial(
        pl.kernel,
        out_type=jax.ShapeDtypeStruct((n_rows, _D), jnp.float32),
        mesh=mesh,
        scratch_types=[
            pltpu.VMEM((b_per_w,), jnp.int32),
            pltpu.VMEM((_NB, _C, _D), jnp.float32),
            pltpu.SemaphoreType.DMA,
            pltpu.SemaphoreType.DMA,
        ],
    )
    def run(idx_hbm, table_hbm, out_hbm, idx_v, rows_v, gsem, osem):
        wid = lax.axis_index("s") * _NC + lax.axis_index("c")
        base = wid * b_per_w
        pltpu.sync_copy(idx_hbm.at[pl.ds(base, b_per_w)], idx_v)

        def g_copy(ch, b):
            return pltpu.make_async_copy(
                table_hbm.at[idx_v.at[pl.ds(ch * _C, _C)]], rows_v.at[b], gsem
            )

        def o_copy(ch, b):
            return pltpu.make_async_copy(
                rows_v.at[b], out_hbm.at[pl.ds(base + ch * _C, _C)], osem
            )

        g_copy(0, 0).start()
        g_copy(1, 1).start()
        for ch in range(4):
            g_copy(ch, ch).wait()
            o_copy(ch, ch).start()
            if ch >= 2:
                o_copy(ch - 2, ch - 2).wait()
            g_copy(ch + 2, (ch + 2) % _NB).start()

        @pl.loop(4, nch, step=_NB)
        def chunk_loop(i):
            for j in range(_NB):
                ch = i + j
                g_copy(ch, j).wait()
                o_copy(ch, j).start()

                @pl.when(ch + 2 < nch)
                def _():
                    o_copy(ch - 2, (j + 2) % _NB).wait()
                    g_copy(ch + 2, (j + 2) % _NB).start()

        for k in range(_NB):
            o_copy(nch - _NB + k, (nch - _NB + k) % _NB).wait()

    return run(idx, table)


def kernel(x, embedding):
    n_samp, width = x.shape
    idx = x.T.reshape(n_samp * width).astype(jnp.int32)
    out = _flat_gather(idx, embedding)
    return out.reshape(width, n_samp, _D).swapaxes(0, 1)


# speedup vs baseline: 1.8326x; 1.0037x over previous
import functools

import jax
import jax.numpy as jnp
from jax import lax
from jax.experimental import pallas as pl
from jax.experimental.pallas import tpu as pltpu
from jax.experimental.pallas import tpu_sc as plsc

_D = 128
_NC = 2
_NS = 16
_NW = _NC * _NS
_C = 160
_NB = 5


@jax.jit
def _flat_gather(idx, table):
    n_rows = idx.shape[0]
    b_per_w = n_rows // _NW
    nch = b_per_w // _C
    mesh = plsc.VectorSubcoreMesh(core_axis_name="c", subcore_axis_name="s")

    @functools.partial(
        pl.kernel,
        out_type=jax.ShapeDtypeStruct((n_rows, _D), jnp.float32),
        mesh=mesh,
        scratch_types=[
            pltpu.VMEM((b_per_w,), jnp.int32),
            pltpu.VMEM((_NB, _C, _D), jnp.float32),
            pltpu.SemaphoreType.DMA,
            pltpu.SemaphoreType.DMA,
        ],
    )
    def run(idx_hbm, table_hbm, out_hbm, idx_v, rows_v, gsem, osem):
        wid = lax.axis_index("s") * _NC + lax.axis_index("c")
        base = wid * b_per_w
        pltpu.sync_copy(idx_hbm.at[pl.ds(base, b_per_w)], idx_v)

        def g_copy(ch, b):
            return pltpu.make_async_copy(
                table_hbm.at[idx_v.at[pl.ds(ch * _C, _C)]], rows_v.at[b], gsem
            )

        def o_copy(ch, b):
            return pltpu.make_async_copy(
                rows_v.at[b], out_hbm.at[pl.ds(base + ch * _C, _C)], osem
            )

        g_copy(0, 0).start()
        g_copy(1, 1).start()

        @pl.loop(0, nch)
        def chunk_loop(ch):
            b = lax.rem(ch, _NB)
            g_copy(ch, b).wait()
            o_copy(ch, b).start()

            @pl.when(ch + 2 < nch)
            def _():
                @pl.when(ch >= _NB - 2)
                def _():
                    o_copy(ch - _NB + 2, lax.rem(ch + 2, _NB)).wait()

                g_copy(ch + 2, lax.rem(ch + 2, _NB)).start()

        @pl.loop(nch - _NB, nch)
        def drain(k):
            o_copy(k, lax.rem(k, _NB)).wait()

    return run(idx, table)


def kernel(x, embedding):
    n_samp, width = x.shape
    idx = x.T.reshape(n_samp * width).astype(jnp.int32)
    out = _flat_gather(idx, embedding)
    return out.reshape(width, n_samp, _D).swapaxes(0, 1)
